# baseline (speedup 1.0000x reference)
import functools

import jax
import jax.numpy as jnp
from jax.experimental import pallas as pl
from jax.experimental.pallas import tpu as pltpu

B = 4
N = 256
TJ = 64
JT = N // TJ
OUT_PAD = 8
NW = 32


def _fused_kernel(*refs):
    r2_ref, pos_ref, posT_ref, vel_ref, goal_ref = refs[:5]
    wbr = (refs[5:5 + NW], refs[5 + NW:5 + 2 * NW])
    out_ref = refs[5 + 2 * NW]
    u_s, ub_s, loc_s, agg_s, deg_s = refs[6 + 2 * NW:]

    jt = pl.program_id(1)

    @pl.when(jt == 0)
    def _prologue():
        pos = pos_ref[0]
        vel = vel_ref[0]
        goal = goal_ref[0]
        x = jnp.concatenate([goal - pos, pos, pos + vel], axis=1)
        for br in range(2):
            (we0, be0, we1, be1, we2, be2, we3, be3,
             wg0, bg0, wg1h, bg1, wg2, bg2,
             wp0, bp0, wp1, bp1, wp2, bp2,
             wl0, bl0, wl1, bl1, wl2, bl2,
             wq0, bq0, wq1, bq1, wq2, bq2) = wbr[br]
            e = jnp.maximum(jnp.dot(x, we0[...], preferred_element_type=jnp.float32) + be0[...], 0.0)
            e = jnp.maximum(jnp.dot(e, we1[...], preferred_element_type=jnp.float32) + be1[...], 0.0)
            e = jnp.maximum(jnp.dot(e, we2[...], preferred_element_type=jnp.float32) + be2[...], 0.0)
            enc = jnp.dot(e, we3[...], preferred_element_type=jnp.float32) + be3[...]
            u = jnp.dot(enc, wg0[...], preferred_element_type=jnp.float32)
            u_s[br] = u.astype(jnp.bfloat16)
            ub_s[br] = (u - bg0[...]).astype(jnp.bfloat16)

            l = jnp.maximum(jnp.dot(x, wl0[...], preferred_element_type=jnp.float32) + bl0[...], 0.0)
            l = jnp.maximum(jnp.dot(l, wl1[...], preferred_element_type=jnp.float32) + bl1[...], 0.0)
            loc_s[br] = jnp.dot(l, wl2[...], preferred_element_type=jnp.float32) + bl2[...]

            agg_s[br] = jnp.zeros((N, 64), jnp.float32)
        deg_s[...] = jnp.zeros((N, 1), jnp.float32)

    pos = pos_ref[0]
    posT = posT_ref[0, 0]
    px_i = pos[:, 0:1]
    py_i = pos[:, 1:2]
    px_j = posT[0:1, :]
    py_j = posT[1:2, :]
    d2 = (px_i - px_j) ** 2 + (py_i - py_j) ** 2
    maskf2 = (d2 <= r2_ref[0, 0]).astype(jnp.float32)
    maskf = jax.lax.broadcast_in_dim(maskf2, (N, TJ, 64), (0, 1))
    deg_s[...] += jnp.sum(maskf2, axis=1, keepdims=True)

    for br in range(2):
        wg1h, bg1 = wbr[br][10], wbr[br][11]
        ub_i = ub_s[br]
        u_j = u_s[br, pl.ds(jt * TJ, TJ), :]
        t = jnp.maximum(u_j[None, :, :] - ub_i[:, None, :], jnp.bfloat16(0.0))
        t2 = t.reshape(N * TJ, 64)
        t2 = jnp.maximum(
            jnp.dot(t2, wg1h[...], preferred_element_type=jnp.float32) + bg1[...], 0.0)
        msg = t2.reshape(N, TJ, 64)
        agg_s[br] += jnp.sum(msg * maskf, axis=1)

    @pl.when(jt == JT - 1)
    def _epilogue():
        deg = deg_s[...] - 1.0
        for br in range(2):
            (we0, be0, we1, be1, we2, be2, we3, be3,
             wg0, bg0, wg1h, bg1, wg2, bg2,
             wp0, bp0, wp1, bp1, wp2, bp2,
             wl0, bl0, wl1, bl1, wl2, bl2,
             wq0, bq0, wq1, bq1, wq2, bq2) = wbr[br]
            self_r1 = jnp.maximum(
                jnp.dot(jnp.maximum(bg0[...], 0.0).astype(jnp.bfloat16), wg1h[...],
                        preferred_element_type=jnp.float32) + bg1[...], 0.0)
            agg = agg_s[br] - self_r1
            h = jnp.dot(agg, wg2[...], preferred_element_type=jnp.float32) + deg * bg2[...]
            h = jnp.maximum(jnp.dot(h, wp0[...], preferred_element_type=jnp.float32) + bp0[...], 0.0)
            h = jnp.maximum(jnp.dot(h, wp1[...], preferred_element_type=jnp.float32) + bp1[...], 0.0)
            h = jnp.dot(h, wp2[...], preferred_element_type=jnp.float32) + bp2[...]
            z = h + loc_s[br]
            z = jnp.maximum(jnp.dot(z, wq0[...], preferred_element_type=jnp.float32) + bq0[...], 0.0)
            z = jnp.maximum(jnp.dot(z, wq1[...], preferred_element_type=jnp.float32) + bq1[...], 0.0)
            out_ref[br, 0] = jnp.dot(z, wq2[...], preferred_element_type=jnp.float32) + bq2[...]


def _flat_branch(p):
    out = []
    for W, b in p['encoder']:
        out += [W, b.reshape(1, -1)]
    (W0, b0), (W1, b1), (W2, b2) = p['gnn']
    out += [W0, b0.reshape(1, -1), W1.astype(jnp.bfloat16), b1.reshape(1, -1),
            W2, b2.reshape(1, -1)]
    for W, b in p['post_gnn']:
        out += [W, b.reshape(1, -1)]
    for W, b in p['local']:
        out += [W, b.reshape(1, -1)]
    for li, (W, b) in enumerate(p['post']):
        if li == len(p['post']) - 1:
            W = jnp.pad(W, ((0, 0), (0, OUT_PAD - W.shape[1])))
            b = jnp.pad(b, (0, OUT_PAD - b.shape[0]))
        out += [W, b.reshape(1, -1)]
    return out


@functools.partial(jax.jit, static_argnames=())
def kernel(pos, vel, goal, params_policy, params_value, comm_range):
    r2 = jnp.asarray(comm_range, jnp.float32).reshape(1, 1) ** 2
    posT = jnp.swapaxes(pos, 1, 2)
    posT = posT.reshape(B, 2, JT, TJ).swapaxes(1, 2)
    weights = _flat_branch(params_policy) + _flat_branch(params_value)

    def wspec(w):
        nd = w.ndim
        return pl.BlockSpec(w.shape, lambda b, jt, _n=nd: (0,) * _n)

    in_specs = [
        pl.BlockSpec((1, 1), lambda b, jt: (0, 0)),
        pl.BlockSpec((1, N, 2), lambda b, jt: (b, 0, 0)),
        pl.BlockSpec((1, 1, 2, TJ), lambda b, jt: (b, jt, 0, 0)),
        pl.BlockSpec((1, N, 2), lambda b, jt: (b, 0, 0)),
        pl.BlockSpec((1, N, 2), lambda b, jt: (b, 0, 0)),
    ] + [wspec(w) for w in weights]

    out = pl.pallas_call(
        _fused_kernel,
        grid=(B, JT),
        in_specs=in_specs,
        out_specs=pl.BlockSpec((2, 1, N, OUT_PAD), lambda b, jt: (0, b, 0, 0)),
        out_shape=jax.ShapeDtypeStruct((2, B, N, OUT_PAD), jnp.float32),
        scratch_shapes=[
            pltpu.VMEM((2, N, 64), jnp.bfloat16),
            pltpu.VMEM((2, N, 64), jnp.bfloat16),
            pltpu.VMEM((2, N, 64), jnp.float32),
            pltpu.VMEM((2, N, 64), jnp.float32),
            pltpu.VMEM((N, 1), jnp.float32),
        ],
    )(r2, pos, posT, vel, goal, *weights)

    outputs = out[0, :, :, :5].reshape(B, N * 5)
    values = out[1, :, :, 0].reshape(B, N)
    return outputs, values


# speedup vs baseline: 1.1272x; 1.1272x over previous
import functools

import jax
import jax.numpy as jnp
from jax.experimental import pallas as pl
from jax.experimental.pallas import tpu as pltpu

B = 4
N = 256
TJ = 128
JT = N // TJ
OUT_PAD = 8
NW = 32


def _fused_kernel(*refs):
    r2_ref, pos_ref, posT_ref, vel_ref, goal_ref = refs[:5]
    wbr = (refs[5:5 + NW], refs[5 + NW:5 + 2 * NW])
    out_ref = refs[5 + 2 * NW]
    u_s, ub_s, loc_s, agg_s, deg_s = refs[6 + 2 * NW:]

    jt = pl.program_id(1)

    @pl.when(jt == 0)
    def _prologue():
        pos = pos_ref[0]
        vel = vel_ref[0]
        goal = goal_ref[0]
        x = jnp.concatenate([goal - pos, pos, pos + vel], axis=1)
        for br in range(2):
            (we0, be0, we1, be1, we2, be2, we3, be3,
             wg0, bg0, wg1h, bg1, wg2, bg2,
             wp0, bp0, wp1, bp1, wp2, bp2,
             wl0, bl0, wl1, bl1, wl2, bl2,
             wq0, bq0, wq1, bq1, wq2, bq2) = wbr[br]
            e = jnp.maximum(jnp.dot(x, we0[...], preferred_element_type=jnp.float32) + be0[...], 0.0)
            e = jnp.maximum(jnp.dot(e, we1[...], preferred_element_type=jnp.float32) + be1[...], 0.0)
            e = jnp.maximum(jnp.dot(e, we2[...], preferred_element_type=jnp.float32) + be2[...], 0.0)
            enc = jnp.dot(e, we3[...], preferred_element_type=jnp.float32) + be3[...]
            u = jnp.dot(enc, wg0[...], preferred_element_type=jnp.float32)
            u_s[br] = u
            ub_s[br] = u - bg0[...]

            l = jnp.maximum(jnp.dot(x, wl0[...], preferred_element_type=jnp.float32) + bl0[...], 0.0)
            l = jnp.maximum(jnp.dot(l, wl1[...], preferred_element_type=jnp.float32) + bl1[...], 0.0)
            loc_s[br] = jnp.dot(l, wl2[...], preferred_element_type=jnp.float32) + bl2[...]

            agg_s[br] = jnp.zeros((N, 64), jnp.float32)
        deg_s[...] = jnp.zeros((N, 1), jnp.float32)

    pos = pos_ref[0]
    posT = posT_ref[0, 0]
    px_i = pos[:, 0:1]
    py_i = pos[:, 1:2]
    px_j = posT[0:1, :]
    py_j = posT[1:2, :]
    d2 = (px_i - px_j) ** 2 + (py_i - py_j) ** 2
    maskf2 = (d2 <= r2_ref[0, 0]).astype(jnp.float32)
    maskf = jax.lax.broadcast_in_dim(maskf2, (N, TJ, 64), (0, 1))
    deg_s[...] += jnp.sum(maskf2, axis=1, keepdims=True)

    for br in range(2):
        wg1h, bg1 = wbr[br][10], wbr[br][11]
        ub_i = ub_s[br]
        u_j = u_s[br, pl.ds(jt * TJ, TJ), :]
        t = jnp.maximum(u_j[None, :, :] - ub_i[:, None, :], 0.0)
        t2 = t.reshape(N * TJ, 64).astype(jnp.bfloat16)
        t2 = jnp.maximum(
            jnp.dot(t2, wg1h[...], preferred_element_type=jnp.float32) + bg1[...], 0.0)
        msg = t2.reshape(N, TJ, 64)
        agg_s[br] += jnp.sum(msg * maskf, axis=1)

    @pl.when(jt == JT - 1)
    def _epilogue():
        deg = deg_s[...] - 1.0
        for br in range(2):
            (we0, be0, we1, be1, we2, be2, we3, be3,
             wg0, bg0, wg1h, bg1, wg2, bg2,
             wp0, bp0, wp1, bp1, wp2, bp2,
             wl0, bl0, wl1, bl1, wl2, bl2,
             wq0, bq0, wq1, bq1, wq2, bq2) = wbr[br]
            self_r1 = jnp.maximum(
                jnp.dot(jnp.maximum(bg0[...], 0.0).astype(jnp.bfloat16), wg1h[...],
                        preferred_element_type=jnp.float32) + bg1[...], 0.0)
            agg = agg_s[br] - self_r1
            h = jnp.dot(agg, wg2[...], preferred_element_type=jnp.float32) + deg * bg2[...]
            h = jnp.maximum(jnp.dot(h, wp0[...], preferred_element_type=jnp.float32) + bp0[...], 0.0)
            h = jnp.maximum(jnp.dot(h, wp1[...], preferred_element_type=jnp.float32) + bp1[...], 0.0)
            h = jnp.dot(h, wp2[...], preferred_element_type=jnp.float32) + bp2[...]
            z = h + loc_s[br]
            z = jnp.maximum(jnp.dot(z, wq0[...], preferred_element_type=jnp.float32) + bq0[...], 0.0)
            z = jnp.maximum(jnp.dot(z, wq1[...], preferred_element_type=jnp.float32) + bq1[...], 0.0)
            out_ref[br, 0] = jnp.dot(z, wq2[...], preferred_element_type=jnp.float32) + bq2[...]


def _flat_branch(p):
    out = []
    for W, b in p['encoder']:
        out += [W, b.reshape(1, -1)]
    (W0, b0), (W1, b1), (W2, b2) = p['gnn']
    out += [W0, b0.reshape(1, -1), W1.astype(jnp.bfloat16), b1.reshape(1, -1),
            W2, b2.reshape(1, -1)]
    for W, b in p['post_gnn']:
        out += [W, b.reshape(1, -1)]
    for W, b in p['local']:
        out += [W, b.reshape(1, -1)]
    for li, (W, b) in enumerate(p['post']):
        if li == len(p['post']) - 1:
            W = jnp.pad(W, ((0, 0), (0, OUT_PAD - W.shape[1])))
            b = jnp.pad(b, (0, OUT_PAD - b.shape[0]))
        out += [W, b.reshape(1, -1)]
    return out


@functools.partial(jax.jit, static_argnames=())
def kernel(pos, vel, goal, params_policy, params_value, comm_range):
    r2 = jnp.asarray(comm_range, jnp.float32).reshape(1, 1) ** 2
    posT = jnp.swapaxes(pos, 1, 2)
    posT = posT.reshape(B, 2, JT, TJ).swapaxes(1, 2)
    weights = _flat_branch(params_policy) + _flat_branch(params_value)

    def wspec(w):
        nd = w.ndim
        return pl.BlockSpec(w.shape, lambda b, jt, _n=nd: (0,) * _n)

    in_specs = [
        pl.BlockSpec((1, 1), lambda b, jt: (0, 0)),
        pl.BlockSpec((1, N, 2), lambda b, jt: (b, 0, 0)),
        pl.BlockSpec((1, 1, 2, TJ), lambda b, jt: (b, jt, 0, 0)),
        pl.BlockSpec((1, N, 2), lambda b, jt: (b, 0, 0)),
        pl.BlockSpec((1, N, 2), lambda b, jt: (b, 0, 0)),
    ] + [wspec(w) for w in weights]

    out = pl.pallas_call(
        _fused_kernel,
        grid=(B, JT),
        in_specs=in_specs,
        out_specs=pl.BlockSpec((2, 1, N, OUT_PAD), lambda b, jt: (0, b, 0, 0)),
        out_shape=jax.ShapeDtypeStruct((2, B, N, OUT_PAD), jnp.float32),
        scratch_shapes=[
            pltpu.VMEM((2, N, 64), jnp.float32),
            pltpu.VMEM((2, N, 64), jnp.float32),
            pltpu.VMEM((2, N, 64), jnp.float32),
            pltpu.VMEM((2, N, 64), jnp.float32),
            pltpu.VMEM((N, 1), jnp.float32),
        ],
    )(r2, pos, posT, vel, goal, *weights)

    outputs = out[0, :, :, :5].reshape(B, N * 5)
    values = out[1, :, :, 0].reshape(B, N)
    return outputs, values
